# baseline (speedup 1.0000x reference)
import functools
import math

import jax
import jax.numpy as jnp
from jax import lax
from jax.experimental import pallas as pl
from jax.experimental.pallas import tpu as pltpu
from jax.experimental.pallas import tpu_sc as plsc


def kernel(x, pos_encoding):
    b, l, d = x.shape
    scale = math.sqrt(d)
    nw = 32
    rows_w = l // nw
    chunk_rows = 16
    nchunks = rows_w // chunk_rows
    welems = rows_w * d
    celems = chunk_rows * d
    nvec = celems // 16

    xf = x.reshape(b * l * d)
    pef = pos_encoding[:l].reshape(l * d)

    mesh = plsc.VectorSubcoreMesh(core_axis_name="c", subcore_axis_name="s")

    @functools.partial(
        pl.kernel, mesh=mesh,
        out_type=jax.ShapeDtypeStruct((b * l * d,), jnp.float32),
        scratch_types=[
            pltpu.VMEM((welems,), jnp.float32),
            pltpu.VMEM((celems,), jnp.float32),
        ],
    )
    def k(x_hbm, pe_hbm, o_hbm, pe_v, x_v):
        w = lax.axis_index("s") * 2 + lax.axis_index("c")
        pbase = w * welems
        pltpu.sync_copy(pe_hbm.at[pl.ds(pbase, welems)], pe_v)

        def per_chunk(c, carry):
            bi = c // nchunks
            h = c % nchunks
            off = bi * (l * d) + pbase + h * celems
            pltpu.sync_copy(x_hbm.at[pl.ds(off, celems)], x_v)

            def inner(i, carry2):
                s = pl.ds(i * 16, 16)
                xv = x_v[s]
                pv = pe_v[pl.ds(h * celems + i * 16, 16)]
                x_v[s] = jnp.where(xv == 0.0, 0.0, xv * scale + pv)
                return carry2

            lax.fori_loop(0, nvec, inner, 0)
            pltpu.sync_copy(x_v, o_hbm.at[pl.ds(off, celems)])
            return carry

        lax.fori_loop(0, b * nchunks, per_chunk, 0)

    return k(xf, pef).reshape(b, l, d)


# speedup vs baseline: 9.6369x; 9.6369x over previous
import math

import jax
import jax.numpy as jnp
from jax.experimental import pallas as pl


def kernel(x, pos_encoding):
    b, l, d = x.shape
    scale = math.sqrt(d)

    blk = 512
    while l % blk:
        blk //= 2
    nsb = l // blk

    pe = pos_encoding[:l] if pos_encoding.shape[0] != l else pos_encoding

    def body(x_ref, pe_ref, o_ref):
        i = pl.program_id(0)
        xv = x_ref[...]
        peb = pe_ref[pl.ds(i * blk, blk), :]
        o_ref[...] = jnp.where(xv == 0.0, 0.0, xv * scale + peb[None])

    return pl.pallas_call(
        body,
        grid=(nsb,),
        in_specs=[
            pl.BlockSpec((b, blk, d), lambda i: (0, i, 0)),
            pl.BlockSpec((l, d), lambda i: (0, 0)),
        ],
        out_specs=pl.BlockSpec((b, blk, d), lambda i: (0, i, 0)),
        out_shape=jax.ShapeDtypeStruct((b, l, d), x.dtype),
    )(x, pe)
